# Initial kernel scaffold; baseline (speedup 1.0000x reference)
import jax
import jax.numpy as jnp
from jax.experimental import pallas as pl


def kernel(x, start_token, end_token, tok_table, pos_table):
    raise NotImplementedError("write your pallas kernel here")


# speedup vs baseline: 2.4311x; 2.4311x over previous
import functools

import jax
import jax.numpy as jnp
from jax import lax
from jax.experimental import pallas as pl
from jax.experimental.pallas import tpu as pltpu
from jax.experimental.pallas import tpu_sc as plsc

B, T, V, D = 4096, 200, 1000, 64
L = 16
NW = 32
CHUNK = 400
ROWS_PER_W = (B * T) // NW
N_CHUNKS = ROWS_PER_W // CHUNK


def _sc_embed(x_flat, aug_table, pos_block):
    mesh = plsc.VectorSubcoreMesh(core_axis_name="c", subcore_axis_name="s")

    @functools.partial(
        pl.kernel,
        mesh=mesh,
        compiler_params=pltpu.CompilerParams(use_tc_tiling_on_sc=False),
        out_type=jax.ShapeDtypeStruct((B * T, D), jnp.float32),
        scratch_types=[
            pltpu.VMEM((CHUNK,), jnp.int32),
            pltpu.VMEM((CHUNK, D), jnp.float32),
            pltpu.VMEM((CHUNK, D), jnp.float32),
            pltpu.SemaphoreType.DMA,
        ],
    )
    def k(x_hbm, aug_hbm, pos_hbm, out_hbm, idx_v, rows_v, pos_v, sem):
        wid = lax.axis_index("s") * 2 + lax.axis_index("c")
        w_base = wid * ROWS_PER_W

        pltpu.sync_copy(pos_hbm, pos_v)

        def chunk_body(c, carry):
            base = w_base + c * CHUNK
            pltpu.sync_copy(x_hbm.at[pl.ds(base, CHUNK)], idx_v)

            for kk in range(CHUNK // L):
                v = idx_v[pl.ds(kk * L, L)]
                r = lax.iota(jnp.int32, L) + (kk * L)
                p = jnp.where(r >= T, r - T, r)
                idx_v[pl.ds(kk * L, L)] = jnp.where(v == 2, p + V, v)

            pltpu.async_copy(aug_hbm.at[idx_v], rows_v, sem).wait()

            def add_row(i, carry2):
                for j in range(D // L):
                    sl = pl.ds(j * L, L)
                    rows_v[i, sl] = rows_v[i, sl] + pos_v[i, sl]
                return carry2

            lax.fori_loop(0, CHUNK, add_row, 0, unroll=2)

            pltpu.sync_copy(rows_v, out_hbm.at[pl.ds(base, CHUNK)])
            return carry

        lax.fori_loop(0, N_CHUNKS, chunk_body, 0)

    return k(x_flat, aug_table, pos_block)


def kernel(x, start_token, end_token, tok_table, pos_table):
    x_flat = x.reshape(-1)
    aug_table = jnp.concatenate(
        [tok_table, jnp.float32(-5.0) - pos_table], axis=0)
    pos_block = jnp.concatenate([pos_table, pos_table], axis=0)
    out = _sc_embed(x_flat, aug_table, pos_block)
    return out.reshape(B, T, D)


# speedup vs baseline: 2.9894x; 1.2296x over previous
import functools

import jax
import jax.numpy as jnp
from jax import lax
from jax.experimental import pallas as pl
from jax.experimental.pallas import tpu as pltpu
from jax.experimental.pallas import tpu_sc as plsc

B, T, V, D = 4096, 200, 1000, 64
L = 16
NW = 32
CHUNK = 200
NBUF = 4
ROWS_PER_W = (B * T) // NW
N_CHUNKS = ROWS_PER_W // CHUNK
N_VREG = ROWS_PER_W // L


def _sc_embed(x_flat, aug_table, pos_table):
    mesh = plsc.VectorSubcoreMesh(core_axis_name="c", subcore_axis_name="s")

    @functools.partial(
        pl.kernel,
        mesh=mesh,
        compiler_params=pltpu.CompilerParams(use_tc_tiling_on_sc=False),
        out_type=jax.ShapeDtypeStruct((B * T, D), jnp.float32),
        scratch_types=(
            [pltpu.VMEM((ROWS_PER_W,), jnp.int32)]
            + [pltpu.VMEM((CHUNK, D), jnp.float32) for _ in range(NBUF)]
            + [pltpu.VMEM((CHUNK, D), jnp.float32)]
            + [pltpu.SemaphoreType.DMA for _ in range(2 * NBUF)]
        ),
    )
    def k(x_hbm, aug_hbm, pos_hbm, out_hbm, idx_v, r0, r1, r2, r3, pos_v,
          g0, g1, g2, g3, o0, o1, o2, o3):
        rows = (r0, r1, r2, r3)
        gsem = (g0, g1, g2, g3)
        osem = (o0, o1, o2, o3)
        wid = lax.axis_index("s") * 2 + lax.axis_index("c")
        w_base = wid * ROWS_PER_W

        pltpu.sync_copy(pos_hbm, pos_v)
        pltpu.sync_copy(x_hbm.at[pl.ds(w_base, ROWS_PER_W)], idx_v)

        iota = lax.iota(jnp.int32, L)

        def remap(j, carry):
            m = (j * L) % T
            r = iota + m
            p = jnp.where(r >= T, r - T, r)
            v = idx_v[pl.ds(j * L, L)]
            idx_v[pl.ds(j * L, L)] = jnp.where(v == 2, p + V, v)
            return carry

        lax.fori_loop(0, N_VREG, remap, 0, unroll=8)

        def g_desc(c, b):
            return pltpu.make_async_copy(
                aug_hbm.at[idx_v.at[pl.ds(c * CHUNK, CHUNK)]],
                rows[b], gsem[b])

        def o_desc(c, b):
            return pltpu.make_async_copy(
                rows[b], out_hbm.at[pl.ds(w_base + c * CHUNK, CHUNK)],
                osem[b])

        for b in range(NBUF - 1):
            g_desc(b, b).start()

        def add_rows(buf):
            def add_row(i, carry):
                for j in range(D // L):
                    sl = (i, pl.ds(j * L, L))
                    buf[sl] = buf[sl] + pos_v[sl]
                return carry
            lax.fori_loop(0, CHUNK, add_row, 0, unroll=2)

        def step(c, b):
            g_desc(c, b).wait()
            add_rows(rows[b])
            o_desc(c, b).start()
            cn = c + NBUF - 1
            bn = (b + NBUF - 1) % NBUF

            @pl.when(cn < N_CHUNKS)
            def _():
                @pl.when(c >= 1)
                def _():
                    o_desc(c - 1, bn).wait()
                g_desc(cn, bn).start()

        def outer(g, carry):
            c0 = g * NBUF
            for b in range(NBUF):
                step(c0 + b, b)
            return carry

        lax.fori_loop(0, N_CHUNKS // NBUF, outer, 0)

        for b in range(NBUF):
            o_desc(N_CHUNKS - NBUF + b, b).wait()

    return k(x_flat, aug_table, pos_table)


def kernel(x, start_token, end_token, tok_table, pos_table):
    x_flat = x.reshape(-1)
    aug_table = jnp.concatenate(
        [tok_table, jnp.float32(-5.0) - pos_table], axis=0)
    out = _sc_embed(x_flat, aug_table, pos_table)
    return out.reshape(B, T, D)
